# baseline (speedup 1.0000x reference)
import functools

import jax
import jax.numpy as jnp
from jax import lax
from jax.experimental import pallas as pl
from jax.experimental.pallas import tpu as pltpu
from jax.experimental.pallas import tpu_sc as plsc

N = 10000
D = 128
E = 320000
NC = 2
NS = 16
NW = NC * NS
EB = 128
NB = 80
NB0 = 80
NB1 = 80
NBH = 40
EC0 = NS * NB0 * EB
EPAD = NW * NB * EB
NPAD = 10240
RPT = NPAD // NS
DW = 128

_eps = 1e-5


def _fill(ref, nrows, ncols, value):
    def body(i, _):
        r = i // (ncols // 16)
        c = (i % (ncols // 16)) * 16
        ref[r, pl.ds(c, 16)] = jnp.full((16,), value, jnp.float32)
        return 0
    lax.fori_loop(0, nrows * (ncols // 16), body, 0)


def _sc_body(table_hbm, srcp0_hbm, dstp0_hbm, srcp1_hbm, dstp1_hbm, out_hbm,
             sidx, didx, rows, rows1, acc, sem, sem1, sem_s, sem_s1):
    cid = lax.axis_index("c")
    sid = lax.axis_index("s")
    row0 = sid * RPT

    _fill(rows, EB, D, 0.0)
    for k in range(RPT // EB):
        pltpu.sync_copy(rows, acc.at[pl.ds(row0 + k * EB, EB)])
    plsc.subcore_barrier()

    def _run(src_hbm, dst_hbm, nh):
        for h in range(nh):
            pltpu.sync_copy(src_hbm.at[sid, pl.ds(h * NBH, NBH)], sidx)
            pltpu.sync_copy(dst_hbm.at[sid, pl.ds(h * NBH, NBH)], didx)
            pltpu.async_copy(table_hbm.at[sidx.at[0]], rows, sem)

            def _step(t, _):
                j0 = 2 * t
                pltpu.make_async_copy(table_hbm.at[sidx.at[j0]], rows, sem).wait()
                pltpu.async_copy(rows, acc.at[didx.at[j0]], sem_s, add=True)

                @pl.when(t > 0)
                def _():
                    pltpu.make_async_copy(
                        rows1, acc.at[didx.at[j0 - 1]], sem_s1).wait()
                pltpu.async_copy(table_hbm.at[sidx.at[j0 + 1]], rows1, sem1)
                pltpu.make_async_copy(table_hbm.at[sidx.at[j0 + 1]], rows1, sem1).wait()
                pltpu.async_copy(rows1, acc.at[didx.at[j0 + 1]], sem_s1, add=True)
                pltpu.make_async_copy(rows, acc.at[didx.at[j0]], sem_s).wait()

                @pl.when(j0 + 2 < NBH)
                def _():
                    pltpu.async_copy(table_hbm.at[sidx.at[j0 + 2]], rows, sem)
                return 0
            lax.fori_loop(0, NBH // 2, _step, 0)
            pltpu.make_async_copy(rows1, acc.at[didx.at[NBH - 1]], sem_s1).wait()

    @pl.when(cid == 0)
    def _():
        _run(srcp0_hbm, dstp0_hbm, NB0 // NBH)

    @pl.when(cid == 1)
    def _():
        _run(srcp1_hbm, dstp1_hbm, NB1 // NBH)
    plsc.subcore_barrier()

    pltpu.sync_copy(acc.at[pl.ds(row0, RPT)], out_hbm.at[cid, pl.ds(row0, RPT)])


_sc_scatter = functools.partial(
    pl.kernel,
    out_type=jax.ShapeDtypeStruct((NC, NPAD, D), jnp.float32),
    mesh=plsc.VectorSubcoreMesh(core_axis_name="c", subcore_axis_name="s"),
    scratch_types=[
        pltpu.VMEM((NBH, EB), jnp.int32),
        pltpu.VMEM((NBH, EB), jnp.int32),
        pltpu.VMEM((EB, D), jnp.float32),
        pltpu.VMEM((EB, D), jnp.float32),
        pltpu.VMEM_SHARED((NPAD, D), jnp.float32),
        pltpu.SemaphoreType.DMA,
        pltpu.SemaphoreType.DMA,
        pltpu.SemaphoreType.DMA,
        pltpu.SemaphoreType.DMA,
    ],
)(_sc_body)


def _split_edges(padded):
    p0 = padded[:EC0].reshape(NS, NB0, EB)
    if NB1 == 0:
        p1 = padded[:EB].reshape(NS // NS, 1, EB) * 0 + (NPAD - EB)
        p1 = jnp.broadcast_to(p1, (NS, 1, EB))
    else:
        p1 = padded[EC0:].reshape(NS, NB1, EB)
    return p0, p1


def _sc_deg_body(dstp_hbm, out_hbm, didx, zbuf, obuf, acc):
    cid = lax.axis_index("c")
    sid = lax.axis_index("s")
    wid = cid * NS + sid
    row0 = sid * RPT

    pltpu.sync_copy(dstp_hbm.at[wid], didx)
    _fill(zbuf, EB, DW, 0.0)
    _fill(obuf, EB, DW, 1.0)
    for k in range(RPT // EB):
        pltpu.sync_copy(zbuf, acc.at[pl.ds(row0 + k * EB, EB)])
    plsc.subcore_barrier()

    def _step(j, _):
        pltpu.sync_copy(obuf, acc.at[didx.at[j]], add=True)
        return 0
    lax.fori_loop(0, NB, _step, 0)
    plsc.subcore_barrier()

    pltpu.sync_copy(acc.at[pl.ds(row0, RPT)], out_hbm.at[cid, pl.ds(row0, RPT)])


_sc_deg = functools.partial(
    pl.kernel,
    out_type=jax.ShapeDtypeStruct((NC, NPAD, DW), jnp.float32),
    mesh=plsc.VectorSubcoreMesh(core_axis_name="c", subcore_axis_name="s"),
    scratch_types=[
        pltpu.VMEM((NB, EB), jnp.int32),
        pltpu.VMEM((EB, DW), jnp.float32),
        pltpu.VMEM((EB, DW), jnp.float32),
        pltpu.VMEM_SHARED((NPAD, DW), jnp.float32),
    ],
)(_sc_deg_body)


BR = 256


def _dinv_of(p0d, p1d):
    deg = p0d[:, :1] + p1d[:, :1] + 1.0
    return lax.rsqrt(deg)


def _tc_pre_body(x_ref, w_ref, p0d_ref, p1d_ref, hp_ref):
    dinv = _dinv_of(p0d_ref[...], p1d_ref[...])
    h = jnp.dot(x_ref[...], w_ref[...], preferred_element_type=jnp.float32)
    hp_ref[...] = h * dinv


def _post_math(hp, p0, p1, dinv, b, g, bt):
    agg = dinv * (p0 + p1 + hp) + b[None, :]
    mu = jnp.mean(agg, axis=-1, keepdims=True)
    var = jnp.mean((agg - mu) ** 2, axis=-1, keepdims=True)
    y = (agg - mu) * lax.rsqrt(var + _eps) * g[None, :] + bt[None, :]
    return jnp.maximum(y, 0.0)


def _tc_mid_body(residual, hp_ref, p0_ref, p1_ref, p0d_ref, p1d_ref,
                 xprev_ref, b_ref, g_ref, bt_ref, wn_ref, x_ref, hpn_ref):
    dinv = _dinv_of(p0d_ref[...], p1d_ref[...])
    y = _post_math(hp_ref[...], p0_ref[...], p1_ref[...], dinv,
                   b_ref[...], g_ref[...], bt_ref[...])
    if residual:
        y = y + xprev_ref[...]
    x_ref[...] = y
    hpn_ref[...] = jnp.dot(y, wn_ref[...], preferred_element_type=jnp.float32) * dinv


def _tc_final_body(hp_ref, p0_ref, p1_ref, p0d_ref, p1d_ref,
                   xprev_ref, b_ref, g_ref, bt_ref, x_ref):
    dinv = _dinv_of(p0d_ref[...], p1d_ref[...])
    y = _post_math(hp_ref[...], p0_ref[...], p1_ref[...], dinv,
                   b_ref[...], g_ref[...], bt_ref[...])
    x_ref[...] = y + xprev_ref[...]


_row_spec = pl.BlockSpec((BR, D), lambda i: (i, 0))
_deg_spec = _row_spec
_full_spec = pl.BlockSpec((D, D), lambda i: (0, 0))
_vec_spec = pl.BlockSpec((D,), lambda i: (0,))
_grid = (NPAD // BR,)
_out_rows = jax.ShapeDtypeStruct((NPAD, D), jnp.float32)

_tc_pre = pl.pallas_call(
    _tc_pre_body,
    grid=_grid,
    in_specs=[_row_spec, _full_spec, _deg_spec, _deg_spec],
    out_specs=_row_spec,
    out_shape=_out_rows,
)

_tc_mid = [
    pl.pallas_call(
        functools.partial(_tc_mid_body, residual),
        grid=_grid,
        in_specs=[_row_spec, _row_spec, _row_spec, _deg_spec, _deg_spec,
                  _row_spec, _vec_spec, _vec_spec, _vec_spec, _full_spec],
        out_specs=[_row_spec, _row_spec],
        out_shape=[_out_rows, _out_rows],
    )
    for residual in (False, True)
]

_tc_final = pl.pallas_call(
    _tc_final_body,
    grid=_grid,
    in_specs=[_row_spec, _row_spec, _row_spec, _deg_spec, _deg_spec,
              _row_spec, _vec_spec, _vec_spec, _vec_spec],
    out_specs=_row_spec,
    out_shape=_out_rows,
)


@jax.jit
def kernel(x, edge_index, W0, b0, g0, bt0, W1, b1, g1, bt1, W2, b2, g2, bt2):
    src = edge_index[0].astype(jnp.int32)
    dst = edge_index[1].astype(jnp.int32)
    pad_ar = jnp.arange(EPAD - E, dtype=jnp.int32)
    trash = N + pad_ar % (NPAD - N)
    src_full = jnp.concatenate([src, (pad_ar * 797) % N])
    dst_full = jnp.concatenate([dst, trash])
    srcp0, srcp1 = _split_edges(src_full)
    dstp0, dstp1 = _split_edges(dst_full)
    dstp = dst_full.reshape(NW, NB, EB)
    xp = jnp.pad(x, ((0, NPAD - N), (0, 0)))

    degp = _sc_deg(dstp)
    p0d, p1d = degp[0], degp[1]

    hp = _tc_pre(xp, W0, p0d, p1d)
    params = [(b0, g0, bt0), (b1, g1, bt1), (b2, g2, bt2)]
    Wn = [W1, W2]
    xcur = xp
    for i in range(2):
        parts = _sc_scatter(hp, srcp0, dstp0, srcp1, dstp1)
        b, g, bt = params[i]
        xcur, hp = _tc_mid[1 if i > 0 else 0](
            hp, parts[0], parts[1], p0d, p1d, xcur, b, g, bt, Wn[i])
    parts = _sc_scatter(hp, srcp0, dstp0, srcp1, dstp1)
    b, g, bt = params[2]
    out = _tc_final(hp, parts[0], parts[1], p0d, p1d, xcur, b, g, bt)
    return out[:N]


# speedup vs baseline: 1.0870x; 1.0870x over previous
import functools

import jax
import jax.numpy as jnp
from jax import lax
from jax.experimental import pallas as pl
from jax.experimental.pallas import tpu as pltpu
from jax.experimental.pallas import tpu_sc as plsc

N = 10000
D = 128
E = 320000
NC = 2
NS = 16
NW = NC * NS
EB = 128
NB = 80
NB0 = 80
NB1 = 80
NBH = 40
EC0 = NS * NB0 * EB
EPAD = NW * NB * EB
NPAD = 10240
RPT = NPAD // NS
DW = 128

_eps = 1e-5


def _fill(ref, nrows, ncols, value):
    def body(i, _):
        r = i // (ncols // 16)
        c = (i % (ncols // 16)) * 16
        ref[r, pl.ds(c, 16)] = jnp.full((16,), value, jnp.float32)
        return 0
    lax.fori_loop(0, nrows * (ncols // 16), body, 0)


def _sc_body(table_hbm, srcp0_hbm, dstp0_hbm, srcp1_hbm, dstp1_hbm, out_hbm,
             sidx, didx, rows, rows1, acc, sem, sem1):
    cid = lax.axis_index("c")
    sid = lax.axis_index("s")
    row0 = sid * RPT

    _fill(rows, EB, D, 0.0)
    for k in range(RPT // EB):
        pltpu.sync_copy(rows, acc.at[pl.ds(row0 + k * EB, EB)])
    plsc.subcore_barrier()

    def _run(src_hbm, dst_hbm, nh):
        for h in range(nh):
            pltpu.sync_copy(src_hbm.at[sid, pl.ds(h * NBH, NBH)], sidx)
            pltpu.sync_copy(dst_hbm.at[sid, pl.ds(h * NBH, NBH)], didx)
            pltpu.async_copy(table_hbm.at[sidx.at[0]], rows, sem)

            def _step(t, _):
                j0 = 2 * t
                pltpu.make_async_copy(table_hbm.at[sidx.at[j0]], rows, sem).wait()
                gather1 = pltpu.async_copy(table_hbm.at[sidx.at[j0 + 1]], rows1, sem1)
                pltpu.sync_copy(rows, acc.at[didx.at[j0]], add=True)

                @pl.when(j0 + 2 < NBH)
                def _():
                    pltpu.async_copy(table_hbm.at[sidx.at[j0 + 2]], rows, sem)
                gather1.wait()
                pltpu.sync_copy(rows1, acc.at[didx.at[j0 + 1]], add=True)
                return 0
            lax.fori_loop(0, NBH // 2, _step, 0)

    @pl.when(cid == 0)
    def _():
        _run(srcp0_hbm, dstp0_hbm, NB0 // NBH)

    @pl.when(cid == 1)
    def _():
        _run(srcp1_hbm, dstp1_hbm, NB1 // NBH)
    plsc.subcore_barrier()

    pltpu.sync_copy(acc.at[pl.ds(row0, RPT)], out_hbm.at[cid, pl.ds(row0, RPT)])


_sc_scatter = functools.partial(
    pl.kernel,
    out_type=jax.ShapeDtypeStruct((NC, NPAD, D), jnp.float32),
    mesh=plsc.VectorSubcoreMesh(core_axis_name="c", subcore_axis_name="s"),
    scratch_types=[
        pltpu.VMEM((NBH, EB), jnp.int32),
        pltpu.VMEM((NBH, EB), jnp.int32),
        pltpu.VMEM((EB, D), jnp.float32),
        pltpu.VMEM((EB, D), jnp.float32),
        pltpu.VMEM_SHARED((NPAD, D), jnp.float32),
        pltpu.SemaphoreType.DMA,
        pltpu.SemaphoreType.DMA,
    ],
)(_sc_body)


def _split_edges(padded):
    p0 = padded[:EC0].reshape(NS, NB0, EB)
    if NB1 == 0:
        p1 = padded[:EB].reshape(NS // NS, 1, EB) * 0 + (NPAD - EB)
        p1 = jnp.broadcast_to(p1, (NS, 1, EB))
    else:
        p1 = padded[EC0:].reshape(NS, NB1, EB)
    return p0, p1


def _sc_deg_body(dstp_hbm, out_hbm, didx, zbuf, obuf, acc):
    cid = lax.axis_index("c")
    sid = lax.axis_index("s")
    wid = cid * NS + sid
    row0 = sid * RPT

    pltpu.sync_copy(dstp_hbm.at[wid], didx)
    _fill(zbuf, EB, DW, 0.0)
    _fill(obuf, EB, DW, 1.0)
    for k in range(RPT // EB):
        pltpu.sync_copy(zbuf, acc.at[pl.ds(row0 + k * EB, EB)])
    plsc.subcore_barrier()

    def _step(j, _):
        pltpu.sync_copy(obuf, acc.at[didx.at[j]], add=True)
        return 0
    lax.fori_loop(0, NB, _step, 0)
    plsc.subcore_barrier()

    pltpu.sync_copy(acc.at[pl.ds(row0, RPT)], out_hbm.at[cid, pl.ds(row0, RPT)])


_sc_deg = functools.partial(
    pl.kernel,
    out_type=jax.ShapeDtypeStruct((NC, NPAD, DW), jnp.float32),
    mesh=plsc.VectorSubcoreMesh(core_axis_name="c", subcore_axis_name="s"),
    scratch_types=[
        pltpu.VMEM((NB, EB), jnp.int32),
        pltpu.VMEM((EB, DW), jnp.float32),
        pltpu.VMEM((EB, DW), jnp.float32),
        pltpu.VMEM_SHARED((NPAD, DW), jnp.float32),
    ],
)(_sc_deg_body)


BR = 256


def _dinv_of(p0d, p1d):
    deg = p0d[:, :1] + p1d[:, :1] + 1.0
    return lax.rsqrt(deg)


def _tc_mm_body(x_ref, w_ref, h_ref):
    h_ref[...] = jnp.dot(x_ref[...], w_ref[...],
                         preferred_element_type=jnp.float32)


def _tc_scale_body(h_ref, p0d_ref, p1d_ref, hp_ref):
    hp_ref[...] = h_ref[...] * _dinv_of(p0d_ref[...], p1d_ref[...])


def _post_math(hp, p0, p1, dinv, b, g, bt):
    agg = dinv * (p0 + p1 + hp) + b[None, :]
    mu = jnp.mean(agg, axis=-1, keepdims=True)
    var = jnp.mean((agg - mu) ** 2, axis=-1, keepdims=True)
    y = (agg - mu) * lax.rsqrt(var + _eps) * g[None, :] + bt[None, :]
    return jnp.maximum(y, 0.0)


def _tc_mid_body(residual, hp_ref, p0_ref, p1_ref, p0d_ref, p1d_ref,
                 xprev_ref, b_ref, g_ref, bt_ref, wn_ref, x_ref, hpn_ref):
    dinv = _dinv_of(p0d_ref[...], p1d_ref[...])
    y = _post_math(hp_ref[...], p0_ref[...], p1_ref[...], dinv,
                   b_ref[...], g_ref[...], bt_ref[...])
    if residual:
        y = y + xprev_ref[...]
    x_ref[...] = y
    hpn_ref[...] = jnp.dot(y, wn_ref[...], preferred_element_type=jnp.float32) * dinv


def _tc_final_body(hp_ref, p0_ref, p1_ref, p0d_ref, p1d_ref,
                   xprev_ref, b_ref, g_ref, bt_ref, x_ref):
    dinv = _dinv_of(p0d_ref[...], p1d_ref[...])
    y = _post_math(hp_ref[...], p0_ref[...], p1_ref[...], dinv,
                   b_ref[...], g_ref[...], bt_ref[...])
    x_ref[...] = y + xprev_ref[...]


_row_spec = pl.BlockSpec((BR, D), lambda i: (i, 0))
_deg_spec = _row_spec
_full_spec = pl.BlockSpec((D, D), lambda i: (0, 0))
_vec_spec = pl.BlockSpec((D,), lambda i: (0,))
_grid = (NPAD // BR,)
_out_rows = jax.ShapeDtypeStruct((NPAD, D), jnp.float32)

_tc_mm = pl.pallas_call(
    _tc_mm_body,
    grid=_grid,
    in_specs=[_row_spec, _full_spec],
    out_specs=_row_spec,
    out_shape=_out_rows,
)

_tc_scale = pl.pallas_call(
    _tc_scale_body,
    grid=_grid,
    in_specs=[_row_spec, _deg_spec, _deg_spec],
    out_specs=_row_spec,
    out_shape=_out_rows,
)

_tc_mid = [
    pl.pallas_call(
        functools.partial(_tc_mid_body, residual),
        grid=_grid,
        in_specs=[_row_spec, _row_spec, _row_spec, _deg_spec, _deg_spec,
                  _row_spec, _vec_spec, _vec_spec, _vec_spec, _full_spec],
        out_specs=[_row_spec, _row_spec],
        out_shape=[_out_rows, _out_rows],
    )
    for residual in (False, True)
]

_tc_final = pl.pallas_call(
    _tc_final_body,
    grid=_grid,
    in_specs=[_row_spec, _row_spec, _row_spec, _deg_spec, _deg_spec,
              _row_spec, _vec_spec, _vec_spec, _vec_spec],
    out_specs=_row_spec,
    out_shape=_out_rows,
)


@jax.jit
def kernel(x, edge_index, W0, b0, g0, bt0, W1, b1, g1, bt1, W2, b2, g2, bt2):
    src = edge_index[0].astype(jnp.int32)
    dst = edge_index[1].astype(jnp.int32)
    pad_ar = jnp.arange(EPAD - E, dtype=jnp.int32)
    trash = N + pad_ar % (NPAD - N)
    src_full = jnp.concatenate([src, (pad_ar * 797) % N])
    dst_full = jnp.concatenate([dst, trash])
    srcp0, srcp1 = _split_edges(src_full)
    dstp0, dstp1 = _split_edges(dst_full)
    dstp = dst_full.reshape(NW, NB, EB)
    xp = jnp.pad(x, ((0, NPAD - N), (0, 0)))

    h0 = _tc_mm(xp, W0)
    degp = _sc_deg(dstp)
    p0d, p1d = degp[0], degp[1]

    hp = _tc_scale(h0, p0d, p1d)
    params = [(b0, g0, bt0), (b1, g1, bt1), (b2, g2, bt2)]
    Wn = [W1, W2]
    xcur = xp
    for i in range(2):
        parts = _sc_scatter(hp, srcp0, dstp0, srcp1, dstp1)
        b, g, bt = params[i]
        xcur, hp = _tc_mid[1 if i > 0 else 0](
            hp, parts[0], parts[1], p0d, p1d, xcur, b, g, bt, Wn[i])
    parts = _sc_scatter(hp, srcp0, dstp0, srcp1, dstp1)
    b, g, bt = params[2]
    out = _tc_final(hp, parts[0], parts[1], p0d, p1d, xcur, b, g, bt)
    return out[:N]
